# baseline (speedup 1.0000x reference)
import jax
import jax.numpy as jnp
from jax.experimental import pallas as pl
from jax.experimental.pallas import tpu as pltpu

_N = 4096
_D = 1024
_DZ = 128
_BLK = 512
_GRID = _N // _BLK
_K = int(_N * (1.0 - 0.1))
_LAMD = 0.1


def _radix_step(p, vb, shift, width, k):
    cands = [p + (jnp.int32(u) << shift) for u in range(1, 2 ** width)]
    cnts = [jnp.sum((vb < c).astype(jnp.int32)) for c in cands]
    t = cnts[0].dtype.type(0)
    for c in cnts:
        t = t + (c < k).astype(jnp.int32)
    return p + (t << shift)


def _tie_take(vb, V, comp, r):
    g, w = vb.shape
    tied = vb == V
    tf = tied.astype(jnp.float32)
    iu = jax.lax.broadcasted_iota(jnp.int32, (w, w), 0)
    ju = jax.lax.broadcasted_iota(jnp.int32, (w, w), 1)
    tri_incl = (iu <= ju).astype(jnp.float32)
    run = jax.lax.dot_general(tf, tri_incl, (((1,), (0,)), ((), ())),
                              preferred_element_type=jnp.float32)
    row_tot = run[:, w - 1:w]
    ig = jax.lax.broadcasted_iota(jnp.int32, (g, g), 0)
    jg = jax.lax.broadcasted_iota(jnp.int32, (g, g), 1)
    tri_strict = (jg < ig).astype(jnp.float32)
    row_prefix = jax.lax.dot_general(tri_strict, row_tot,
                                     (((1,), (0,)), ((), ())),
                                     preferred_element_type=jnp.float32)
    rank = run + row_prefix
    sel = tied & (rank <= jax.lax.convert_element_type(r, jnp.float32))
    return jnp.sum(jnp.where(sel, comp, jnp.float32(0.0)))


def _select_sums(v1, c1, v2, c2, k):
    b1 = jax.lax.bitcast_convert_type(v1, jnp.int32)
    b2 = jax.lax.bitcast_convert_type(v2, jnp.int32)
    p1 = _radix_step(jnp.int32(0), b1, 30, 1, k)
    p2 = _radix_step(jnp.int32(0), b2, 30, 1, k)
    for r in range(10):
        s = 27 - 3 * r
        p1 = _radix_step(p1, b1, s, 3, k)
        p2 = _radix_step(p2, b2, s, 3, k)
    below1 = b1 < p1
    below2 = b2 < p2
    n1 = jnp.sum(below1.astype(jnp.int32))
    n2 = jnp.sum(below2.astype(jnp.int32))
    s1 = (jnp.sum(jnp.where(below1, c1, jnp.float32(0.0)))
          + _tie_take(b1, p1, c1, k - n1))
    s2 = (jnp.sum(jnp.where(below2, c2, jnp.float32(0.0)))
          + _tie_take(b2, p2, c2, k - n2))
    return s1, s2


def _body(xr_ref, x_ref, z_ref, out_ref, a_scr, b_scr, zn_scr):
    i = pl.program_id(0)
    x = x_ref[...]
    d0 = x - xr_ref[0]
    d1 = x - xr_ref[1]
    zb = z_ref[...]
    a_scr[i, :] = jnp.sqrt(jnp.sum(d0 * d0, axis=1))
    b_scr[i, :] = jnp.sqrt(jnp.sum(d1 * d1, axis=1))
    zn_scr[i, :] = jnp.sqrt(jnp.sum(zb * zb, axis=1))

    @pl.when(i == _GRID - 1)
    def _():
        a = a_scr[...]
        b = b_scr[...]
        zn = zn_scr[...]
        s1, s2 = _select_sums(zn, a + b, a, zn, _K)
        out_ref[...] = jnp.full((1, 1), s1 / _K + _LAMD * (s2 / _K),
                                dtype=jnp.float32)


def kernel(xr, x, z):
    out = pl.pallas_call(
        _body,
        grid=(_GRID,),
        in_specs=[
            pl.BlockSpec((2, _BLK, _D), lambda i: (0, i, 0)),
            pl.BlockSpec((_BLK, _D), lambda i: (i, 0)),
            pl.BlockSpec((_BLK, _DZ), lambda i: (i, 0)),
        ],
        out_specs=pl.BlockSpec((1, 1), lambda i: (0, 0)),
        out_shape=jax.ShapeDtypeStruct((1, 1), jnp.float32),
        scratch_shapes=[
            pltpu.VMEM((_GRID, _BLK), jnp.float32),
            pltpu.VMEM((_GRID, _BLK), jnp.float32),
            pltpu.VMEM((_GRID, _BLK), jnp.float32),
        ],
    )(xr, x, z)
    return out[0, 0]


# speedup vs baseline: 1.0065x; 1.0065x over previous
import jax
import jax.numpy as jnp
from jax.experimental import pallas as pl
from jax.experimental.pallas import tpu as pltpu

_N = 4096
_D = 1024
_DZ = 128
_BLK = 512
_GRID = _N // _BLK
_K = int(_N * (1.0 - 0.1))
_LAMD = 0.1


def _radix_step(p, vb, shift, width, k):
    cands = [p + (jnp.int32(u) << shift) for u in range(1, 2 ** width)]
    cnts = [jnp.sum((vb < c).astype(jnp.int32)) for c in cands]
    t = cnts[0].dtype.type(0)
    for c in cnts:
        t = t + (c < k).astype(jnp.int32)
    return p + (t << shift)


def _tie_take(vb, V, comp, r):
    g, w = vb.shape
    tied = vb == V
    tf = tied.astype(jnp.float32)
    iu = jax.lax.broadcasted_iota(jnp.int32, (w, w), 0)
    ju = jax.lax.broadcasted_iota(jnp.int32, (w, w), 1)
    tri_incl = (iu <= ju).astype(jnp.float32)
    run = jax.lax.dot_general(tf, tri_incl, (((1,), (0,)), ((), ())),
                              preferred_element_type=jnp.float32)
    row_tot = run[:, w - 1:w]
    ig = jax.lax.broadcasted_iota(jnp.int32, (g, g), 0)
    jg = jax.lax.broadcasted_iota(jnp.int32, (g, g), 1)
    tri_strict = (jg < ig).astype(jnp.float32)
    row_prefix = jax.lax.dot_general(tri_strict, row_tot,
                                     (((1,), (0,)), ((), ())),
                                     preferred_element_type=jnp.float32)
    rank = run + row_prefix
    sel = tied & (rank <= jax.lax.convert_element_type(r, jnp.float32))
    return jnp.sum(jnp.where(sel, comp, jnp.float32(0.0)))


def _select_sums(v1, c1, v2, c2, k):
    b1 = jax.lax.bitcast_convert_type(v1, jnp.int32)
    b2 = jax.lax.bitcast_convert_type(v2, jnp.int32)
    p1 = _radix_step(jnp.int32(0), b1, 28, 3, k)
    p2 = _radix_step(jnp.int32(0), b2, 28, 3, k)
    for r in range(7):
        s = 24 - 4 * r
        p1 = _radix_step(p1, b1, s, 4, k)
        p2 = _radix_step(p2, b2, s, 4, k)
    below1 = b1 < p1
    below2 = b2 < p2
    n1 = jnp.sum(below1.astype(jnp.int32))
    n2 = jnp.sum(below2.astype(jnp.int32))
    s1 = (jnp.sum(jnp.where(below1, c1, jnp.float32(0.0)))
          + _tie_take(b1, p1, c1, k - n1))
    s2 = (jnp.sum(jnp.where(below2, c2, jnp.float32(0.0)))
          + _tie_take(b2, p2, c2, k - n2))
    return s1, s2


def _body(xr_ref, x_ref, z_ref, out_ref, a_scr, b_scr, zn_scr):
    i = pl.program_id(0)
    x = x_ref[...]
    d0 = x - xr_ref[0]
    d1 = x - xr_ref[1]
    zb = z_ref[...]
    a_scr[i, :] = jnp.sqrt(jnp.sum(d0 * d0, axis=1))
    b_scr[i, :] = jnp.sqrt(jnp.sum(d1 * d1, axis=1))
    zn_scr[i, :] = jnp.sqrt(jnp.sum(zb * zb, axis=1))

    @pl.when(i == _GRID - 1)
    def _():
        a = a_scr[...]
        b = b_scr[...]
        zn = zn_scr[...]
        s1, s2 = _select_sums(zn, a + b, a, zn, _K)
        out_ref[...] = jnp.full((1, 1), s1 / _K + _LAMD * (s2 / _K),
                                dtype=jnp.float32)


def kernel(xr, x, z):
    out = pl.pallas_call(
        _body,
        grid=(_GRID,),
        in_specs=[
            pl.BlockSpec((2, _BLK, _D), lambda i: (0, i, 0)),
            pl.BlockSpec((_BLK, _D), lambda i: (i, 0)),
            pl.BlockSpec((_BLK, _DZ), lambda i: (i, 0)),
        ],
        out_specs=pl.BlockSpec((1, 1), lambda i: (0, 0)),
        out_shape=jax.ShapeDtypeStruct((1, 1), jnp.float32),
        scratch_shapes=[
            pltpu.VMEM((_GRID, _BLK), jnp.float32),
            pltpu.VMEM((_GRID, _BLK), jnp.float32),
            pltpu.VMEM((_GRID, _BLK), jnp.float32),
        ],
    )(xr, x, z)
    return out[0, 0]
